# baseline (speedup 1.0000x reference)
import numpy as np

import jax
import jax.numpy as jnp
from jax import lax
from jax.experimental import pallas as pl
from jax.experimental.pallas import tpu as pltpu
from jax.experimental.pallas import tpu_sc as plsc

B, S, N = 8, 1024, 4096
DIN, DOUT = 512, 256
EPS = 1e-5
M1 = B * S
M2 = B * N
RB = 512
NBLK = N // RB

MASK10 = int(np.int32(~np.int32(1023)))

NC, NS = 2, 16
NW = NC * NS
ROWS_PER_W = M2 // NW
CHUNK = 64
NCHUNK = ROWS_PER_W // CHUNK
NCH_TOT = M2 // CHUNK
WPAD = CHUNK + 16


def _mm_bn_relu_small(x_ref, w_ref, b_ref, g_ref, be_ref, o_ref):
    y = jnp.dot(x_ref[...], w_ref[...], preferred_element_type=jnp.float32)
    y = y + b_ref[...]
    mu = jnp.mean(y, axis=0, keepdims=True)
    d = y - mu
    var = jnp.mean(d * d, axis=0, keepdims=True)
    o_ref[...] = jnp.maximum(g_ref[...] * d * jax.lax.rsqrt(var + EPS) + be_ref[...], 0.0).astype(jnp.bfloat16)


def _knn_body(skip_ref, samp_ref, idx_ref, w_ref):
    b = pl.program_id(0)
    sk = skip_ref[0]
    sp = samp_ref[0]
    dx = sk[:, 0:1] - sp[0:1, :]
    dy = sk[:, 1:2] - sp[1:2, :]
    dz = sk[:, 2:3] - sp[2:3, :]
    d2 = jnp.maximum((dx * dx + dy * dy) + dz * dz, 1e-30)
    lane = jax.lax.broadcasted_iota(jnp.int32, (RB, S), 1)
    key = jax.lax.bitcast_convert_type(
        (jax.lax.bitcast_convert_type(d2, jnp.int32) & MASK10) | lane, jnp.float32)
    ks = []
    for k in range(3):
        mk = jnp.min(key, axis=1, keepdims=True)
        ks.append(mk)
        if k < 2:
            key = jnp.where(key == mk, jnp.float32(jnp.inf), key)
    kbits = [jax.lax.bitcast_convert_type(m, jnp.int32) for m in ks]
    idx = jnp.concatenate([kb & 1023 for kb in kbits], axis=1)
    dists = jnp.concatenate(
        [jax.lax.bitcast_convert_type(kb & MASK10, jnp.float32) for kb in kbits], axis=1)
    recip = 1.0 / (dists + 1e-8)
    w = recip / jnp.sum(recip, axis=1, keepdims=True)
    idx_ref[...] = jnp.transpose((idx + b * S).reshape(RB // CHUNK, CHUNK, 3), (0, 2, 1))
    wt = jnp.transpose(w.reshape(RB // CHUNK, CHUNK, 3), (0, 2, 1))
    w_ref[...] = jnp.concatenate(
        [wt, jnp.zeros((RB // CHUNK, 3, WPAD - CHUNK), jnp.float32)], axis=2)


def _mm2_stats_body(x_ref, w_ref, b_ref, y_ref, s_ref, ss_ref):
    i = pl.program_id(0)
    y = jnp.dot(x_ref[...], w_ref[...], preferred_element_type=jnp.float32)
    y = y + b_ref[...]
    y_ref[...] = y

    @pl.when(i == 0)
    def _():
        s_ref[...] = jnp.zeros_like(s_ref)
        ss_ref[...] = jnp.zeros_like(ss_ref)

    s_ref[...] += jnp.sum(y, axis=0, keepdims=True)
    ss_ref[...] += jnp.sum(y * y, axis=0, keepdims=True)


def _bn_add_body(y_ref, s_ref, ss_ref, g_ref, be_ref, it_ref, o_ref):
    mu = s_ref[...] / M2
    var = ss_ref[...] / M2 - mu * mu
    sk = jnp.maximum(g_ref[...] * (y_ref[...] - mu) * jax.lax.rsqrt(var + EPS) + be_ref[...], 0.0)
    o_ref[...] = it_ref[...].astype(jnp.float32) + sk


def _sc_interp(sf_hbm, idx_hbm, w_hbm, out_hbm,
               idx0_v, idx1_v, w0_v, w1_v,
               g00, g01, g02, g10, g11, g12, o_v,
               s00, s01, s02, s10, s11, s12):
    cid = lax.axis_index("c")
    sid = lax.axis_index("s")
    wid = sid * NC + cid
    cbase = wid * NCHUNK
    idx_slots = (idx0_v, idx1_v)
    w_slots = (w0_v, w1_v)
    g_slots = ((g00, g01, g02), (g10, g11, g12))
    sems = ((s00, s01, s02), (s10, s11, s12))

    def fetch(gc, slot):
        pltpu.sync_copy(idx_hbm.at[gc], idx_slots[slot])
        pltpu.sync_copy(w_hbm.at[gc], w_slots[slot])
        for k in range(3):
            pltpu.make_async_copy(
                sf_hbm.at[idx_slots[slot].at[k]], g_slots[slot][k], sems[slot][k]).start()

    fetch(cbase, 0)

    def pair_body(i, carry):
        for slot in range(2):
            ci = 2 * i + slot
            gc = cbase + ci

            @pl.when(ci + 1 < NCHUNK)
            def _():
                fetch(gc + 1, 1 - slot)

            gs = g_slots[slot]
            for k in range(3):
                pltpu.make_async_copy(
                    sf_hbm.at[idx_slots[slot].at[k]], gs[k], sems[slot][k]).wait()
            ws = w_slots[slot]

            def row_body(r, rc):
                w0 = ws[0, pl.ds(r, 16)][0]
                w1 = ws[1, pl.ds(r, 16)][0]
                w2 = ws[2, pl.ds(r, 16)][0]
                himask = jnp.full((16,), -65536, jnp.int32)

                def halves(x):
                    lo = jax.lax.bitcast_convert_type(
                        jax.lax.shift_left(x, 16), jnp.float32)
                    hi = jax.lax.bitcast_convert_type(x & himask, jnp.float32)
                    return lo, hi

                for c in range(128 // 16):
                    sl = pl.ds(c * 16, 16)
                    l0, h0 = halves(gs[0][r, sl])
                    l1, h1 = halves(gs[1][r, sl])
                    l2, h2 = halves(gs[2][r, sl])
                    accl = l0 * w0 + l1 * w1 + l2 * w2
                    acch = h0 * w0 + h1 * w1 + h2 * w2
                    o_v[r, sl] = (jax.lax.shift_right_logical(
                        jax.lax.bitcast_convert_type(accl, jnp.int32), 16)
                        | (jax.lax.bitcast_convert_type(acch, jnp.int32) & himask))
                return rc

            lax.fori_loop(0, CHUNK, row_body, 0)
            pltpu.sync_copy(o_v, out_hbm.at[pl.ds(gc * CHUNK, CHUNK)])
        return carry

    lax.fori_loop(0, NCHUNK // 2, pair_body, 0)


def kernel(sample_feature, sample_xyz, skip_feature, skip_xyz,
           W1, b1, g1, be1, W2, b2, g2, be2):
    f32 = jnp.float32
    x1 = sample_feature.reshape(M1, DIN)
    x2 = skip_feature.reshape(M2, DOUT)
    w1t = W1.T
    w2t = W2.T
    samp_t = sample_xyz.transpose(0, 2, 1)
    row = lambda v: v.reshape(1, DOUT)

    sf = pl.pallas_call(
        _mm_bn_relu_small,
        out_shape=jax.ShapeDtypeStruct((M1, DOUT), jnp.bfloat16),
    )(x1, w1t, row(b1), row(g1), row(be1))
    sf3 = jax.lax.bitcast_convert_type(sf.reshape(M1, 128, 2), jnp.int32)

    idx, w = pl.pallas_call(
        _knn_body,
        grid=(B, NBLK),
        in_specs=[
            pl.BlockSpec((1, RB, 3), lambda b, j: (b, j, 0)),
            pl.BlockSpec((1, 3, S), lambda b, j: (b, 0, 0)),
        ],
        out_specs=[
            pl.BlockSpec((RB // CHUNK, 3, CHUNK), lambda b, j: ((b * NBLK + j), 0, 0)),
            pl.BlockSpec((RB // CHUNK, 3, WPAD), lambda b, j: ((b * NBLK + j), 0, 0)),
        ],
        out_shape=[
            jax.ShapeDtypeStruct((NCH_TOT, 3, CHUNK), jnp.int32),
            jax.ShapeDtypeStruct((NCH_TOT, 3, WPAD), f32),
        ],
    )(skip_xyz, samp_t)
    idx_c, w_c = idx, w

    y2, s2, ss2 = pl.pallas_call(
        _mm2_stats_body,
        grid=(M2 // 1024,),
        in_specs=[
            pl.BlockSpec((1024, DOUT), lambda i: (i, 0)),
            pl.BlockSpec((DOUT, DOUT), lambda i: (0, 0)),
            pl.BlockSpec((1, DOUT), lambda i: (0, 0)),
        ],
        out_specs=[
            pl.BlockSpec((1024, DOUT), lambda i: (i, 0)),
            pl.BlockSpec((1, DOUT), lambda i: (0, 0)),
            pl.BlockSpec((1, DOUT), lambda i: (0, 0)),
        ],
        out_shape=[
            jax.ShapeDtypeStruct((M2, DOUT), f32),
            jax.ShapeDtypeStruct((1, DOUT), f32),
            jax.ShapeDtypeStruct((1, DOUT), f32),
        ],
    )(x2, w2t, row(b2))

    interp = pl.kernel(
        _sc_interp,
        out_type=jax.ShapeDtypeStruct((M2, 128), jnp.int32),
        mesh=plsc.VectorSubcoreMesh(
            core_axis_name="c", subcore_axis_name="s",
            num_cores=NC, num_subcores=NS),
        scratch_types=[
            pltpu.VMEM((3, CHUNK), jnp.int32),
            pltpu.VMEM((3, CHUNK), jnp.int32),
            pltpu.VMEM((3, WPAD), f32),
            pltpu.VMEM((3, WPAD), f32),
            pltpu.VMEM((CHUNK, 128), jnp.int32),
            pltpu.VMEM((CHUNK, 128), jnp.int32),
            pltpu.VMEM((CHUNK, 128), jnp.int32),
            pltpu.VMEM((CHUNK, 128), jnp.int32),
            pltpu.VMEM((CHUNK, 128), jnp.int32),
            pltpu.VMEM((CHUNK, 128), jnp.int32),
            pltpu.VMEM((CHUNK, 128), jnp.int32),
            pltpu.SemaphoreType.DMA,
            pltpu.SemaphoreType.DMA,
            pltpu.SemaphoreType.DMA,
            pltpu.SemaphoreType.DMA,
            pltpu.SemaphoreType.DMA,
            pltpu.SemaphoreType.DMA,
        ],
    )(sf3, idx_c, w_c)
    interp2d = jax.lax.bitcast_convert_type(interp, jnp.bfloat16).reshape(M2, DOUT)

    out2d = pl.pallas_call(
        _bn_add_body,
        grid=(M2 // 1024,),
        in_specs=[
            pl.BlockSpec((1024, DOUT), lambda i: (i, 0)),
            pl.BlockSpec((1, DOUT), lambda i: (0, 0)),
            pl.BlockSpec((1, DOUT), lambda i: (0, 0)),
            pl.BlockSpec((1, DOUT), lambda i: (0, 0)),
            pl.BlockSpec((1, DOUT), lambda i: (0, 0)),
            pl.BlockSpec((1024, DOUT), lambda i: (i, 0)),
        ],
        out_specs=pl.BlockSpec((1024, DOUT), lambda i: (i, 0)),
        out_shape=jax.ShapeDtypeStruct((M2, DOUT), f32),
    )(y2, s2, ss2, row(g2), row(be2), interp2d)

    return (out2d.reshape(B, N, DOUT), skip_xyz)


# speedup vs baseline: 1.6725x; 1.6725x over previous
import numpy as np

import jax
import jax.numpy as jnp
from jax import lax
from jax.experimental import pallas as pl
from jax.experimental.pallas import tpu as pltpu
from jax.experimental.pallas import tpu_sc as plsc

B, S, N = 8, 1024, 4096
DIN, DOUT = 512, 256
EPS = 1e-5
M1 = B * S
M2 = B * N
RB = 1024
NBLK = N // RB

MASK10 = int(np.int32(~np.int32(1023)))

NC, NS = 2, 16
NW = NC * NS
ROWS_PER_W = M2 // NW
CHUNK = 64
NCHUNK = ROWS_PER_W // CHUNK
NCH_TOT = M2 // CHUNK
WPAD = CHUNK + 16


def _mm_bn_relu_small(x_ref, w_ref, b_ref, g_ref, be_ref, o_ref):
    y = jnp.dot(x_ref[...], w_ref[...], preferred_element_type=jnp.float32)
    y = y + b_ref[...]
    mu = jnp.mean(y, axis=0, keepdims=True)
    d = y - mu
    var = jnp.mean(d * d, axis=0, keepdims=True)
    o_ref[...] = jnp.maximum(g_ref[...] * d * jax.lax.rsqrt(var + EPS) + be_ref[...], 0.0)


def _knn_body(skip_ref, samp_ref, idx_ref, w_ref):
    b = pl.program_id(0)
    sk = skip_ref[0]
    sp = samp_ref[0]
    dx = sk[:, 0:1] - sp[0:1, :]
    dy = sk[:, 1:2] - sp[1:2, :]
    dz = sk[:, 2:3] - sp[2:3, :]
    d2 = jnp.maximum((dx * dx + dy * dy) + dz * dz, 1e-30)
    lane = jax.lax.broadcasted_iota(jnp.int32, (RB, S), 1)
    key = jax.lax.bitcast_convert_type(
        (jax.lax.bitcast_convert_type(d2, jnp.int32) & MASK10) | lane, jnp.float32)
    ks = []
    for k in range(3):
        mk = jnp.min(key, axis=1, keepdims=True)
        ks.append(mk)
        if k < 2:
            key = jnp.where(key == mk, jnp.float32(jnp.inf), key)
    kbits = [jax.lax.bitcast_convert_type(m, jnp.int32) for m in ks]
    idx = jnp.concatenate([kb & 1023 for kb in kbits], axis=1)
    dists = jnp.concatenate(
        [jax.lax.bitcast_convert_type(kb & MASK10, jnp.float32) for kb in kbits], axis=1)
    recip = 1.0 / (dists + 1e-8)
    w = recip / jnp.sum(recip, axis=1, keepdims=True)
    idx_ref[...] = jnp.transpose((idx + b * S).reshape(RB // CHUNK, CHUNK, 3), (0, 2, 1))
    wt = jnp.transpose(w.reshape(RB // CHUNK, CHUNK, 3), (0, 2, 1))
    w_ref[...] = jnp.concatenate(
        [wt, jnp.zeros((RB // CHUNK, 3, WPAD - CHUNK), jnp.float32)], axis=2)


def _mm2_stats_body(x_ref, w_ref, b_ref, y_ref, s_ref, ss_ref):
    i = pl.program_id(0)
    y = jnp.dot(x_ref[...], w_ref[...], preferred_element_type=jnp.float32)
    y = y + b_ref[...]
    y_ref[...] = y

    @pl.when(i == 0)
    def _():
        s_ref[...] = jnp.zeros_like(s_ref)
        ss_ref[...] = jnp.zeros_like(ss_ref)

    s_ref[...] += jnp.sum(y, axis=0, keepdims=True)
    ss_ref[...] += jnp.sum(y * y, axis=0, keepdims=True)


def _bn_add_body(y_ref, s_ref, ss_ref, g_ref, be_ref, it_ref, o_ref):
    mu = s_ref[...] / M2
    var = ss_ref[...] / M2 - mu * mu
    sk = jnp.maximum(g_ref[...] * (y_ref[...] - mu) * jax.lax.rsqrt(var + EPS) + be_ref[...], 0.0)
    o_ref[...] = it_ref[...] + sk


def _sc_interp(sf_hbm, idx_hbm, w_hbm, out_hbm,
               idx0_v, idx1_v, w0_v, w1_v,
               g00, g01, g02, g10, g11, g12, o_v,
               s00, s01, s02, s10, s11, s12):
    cid = lax.axis_index("c")
    sid = lax.axis_index("s")
    wid = sid * NC + cid
    cbase = wid * NCHUNK
    idx_slots = (idx0_v, idx1_v)
    w_slots = (w0_v, w1_v)
    g_slots = ((g00, g01, g02), (g10, g11, g12))
    sems = ((s00, s01, s02), (s10, s11, s12))

    def fetch(gc, slot):
        pltpu.sync_copy(idx_hbm.at[gc], idx_slots[slot])
        pltpu.sync_copy(w_hbm.at[gc], w_slots[slot])
        for k in range(3):
            pltpu.make_async_copy(
                sf_hbm.at[idx_slots[slot].at[k]], g_slots[slot][k], sems[slot][k]).start()

    fetch(cbase, 0)

    def pair_body(i, carry):
        for slot in range(2):
            ci = 2 * i + slot
            gc = cbase + ci

            @pl.when(ci + 1 < NCHUNK)
            def _():
                fetch(gc + 1, 1 - slot)

            gs = g_slots[slot]
            for k in range(3):
                pltpu.make_async_copy(
                    sf_hbm.at[idx_slots[slot].at[k]], gs[k], sems[slot][k]).wait()
            ws = w_slots[slot]

            def row_body(r, rc):
                w0 = ws[0, pl.ds(r, 16)][0]
                w1 = ws[1, pl.ds(r, 16)][0]
                w2 = ws[2, pl.ds(r, 16)][0]
                for c in range(DOUT // 16):
                    sl = pl.ds(c * 16, 16)
                    o_v[r, sl] = gs[0][r, sl] * w0 + gs[1][r, sl] * w1 + gs[2][r, sl] * w2
                return rc

            lax.fori_loop(0, CHUNK, row_body, 0)
            pltpu.sync_copy(o_v, out_hbm.at[pl.ds(gc * CHUNK, CHUNK)])
        return carry

    lax.fori_loop(0, NCHUNK // 2, pair_body, 0)


def kernel(sample_feature, sample_xyz, skip_feature, skip_xyz,
           W1, b1, g1, be1, W2, b2, g2, be2):
    f32 = jnp.float32
    x1 = sample_feature.reshape(M1, DIN)
    x2 = skip_feature.reshape(M2, DOUT)
    w1t = W1.T
    w2t = W2.T
    samp_t = sample_xyz.transpose(0, 2, 1)
    row = lambda v: v.reshape(1, DOUT)

    sf = pl.pallas_call(
        _mm_bn_relu_small,
        out_shape=jax.ShapeDtypeStruct((M1, DOUT), f32),
    )(x1, w1t, row(b1), row(g1), row(be1))

    idx, w = pl.pallas_call(
        _knn_body,
        grid=(B, NBLK),
        in_specs=[
            pl.BlockSpec((1, RB, 3), lambda b, j: (b, j, 0)),
            pl.BlockSpec((1, 3, S), lambda b, j: (b, 0, 0)),
        ],
        out_specs=[
            pl.BlockSpec((RB // CHUNK, 3, CHUNK), lambda b, j: ((b * NBLK + j), 0, 0)),
            pl.BlockSpec((RB // CHUNK, 3, WPAD), lambda b, j: ((b * NBLK + j), 0, 0)),
        ],
        out_shape=[
            jax.ShapeDtypeStruct((NCH_TOT, 3, CHUNK), jnp.int32),
            jax.ShapeDtypeStruct((NCH_TOT, 3, WPAD), f32),
        ],
    )(skip_xyz, samp_t)
    idx_c, w_c = idx, w

    y2, s2, ss2 = pl.pallas_call(
        _mm2_stats_body,
        grid=(M2 // 1024,),
        in_specs=[
            pl.BlockSpec((1024, DOUT), lambda i: (i, 0)),
            pl.BlockSpec((DOUT, DOUT), lambda i: (0, 0)),
            pl.BlockSpec((1, DOUT), lambda i: (0, 0)),
        ],
        out_specs=[
            pl.BlockSpec((1024, DOUT), lambda i: (i, 0)),
            pl.BlockSpec((1, DOUT), lambda i: (0, 0)),
            pl.BlockSpec((1, DOUT), lambda i: (0, 0)),
        ],
        out_shape=[
            jax.ShapeDtypeStruct((M2, DOUT), f32),
            jax.ShapeDtypeStruct((1, DOUT), f32),
            jax.ShapeDtypeStruct((1, DOUT), f32),
        ],
    )(x2, w2t, row(b2))

    interp = pl.kernel(
        _sc_interp,
        out_type=jax.ShapeDtypeStruct((M2, DOUT), f32),
        mesh=plsc.VectorSubcoreMesh(
            core_axis_name="c", subcore_axis_name="s",
            num_cores=NC, num_subcores=NS),
        scratch_types=[
            pltpu.VMEM((3, CHUNK), jnp.int32),
            pltpu.VMEM((3, CHUNK), jnp.int32),
            pltpu.VMEM((3, WPAD), f32),
            pltpu.VMEM((3, WPAD), f32),
            pltpu.VMEM((CHUNK, DOUT), f32),
            pltpu.VMEM((CHUNK, DOUT), f32),
            pltpu.VMEM((CHUNK, DOUT), f32),
            pltpu.VMEM((CHUNK, DOUT), f32),
            pltpu.VMEM((CHUNK, DOUT), f32),
            pltpu.VMEM((CHUNK, DOUT), f32),
            pltpu.VMEM((CHUNK, DOUT), f32),
            pltpu.SemaphoreType.DMA,
            pltpu.SemaphoreType.DMA,
            pltpu.SemaphoreType.DMA,
            pltpu.SemaphoreType.DMA,
            pltpu.SemaphoreType.DMA,
            pltpu.SemaphoreType.DMA,
        ],
    )(sf, idx_c, w_c)

    out2d = pl.pallas_call(
        _bn_add_body,
        grid=(M2 // 1024,),
        in_specs=[
            pl.BlockSpec((1024, DOUT), lambda i: (i, 0)),
            pl.BlockSpec((1, DOUT), lambda i: (0, 0)),
            pl.BlockSpec((1, DOUT), lambda i: (0, 0)),
            pl.BlockSpec((1, DOUT), lambda i: (0, 0)),
            pl.BlockSpec((1, DOUT), lambda i: (0, 0)),
            pl.BlockSpec((1024, DOUT), lambda i: (i, 0)),
        ],
        out_specs=pl.BlockSpec((1024, DOUT), lambda i: (i, 0)),
        out_shape=jax.ShapeDtypeStruct((M2, DOUT), f32),
    )(y2, s2, ss2, row(g2), row(be2), interp)

    return (out2d.reshape(B, N, DOUT), skip_xyz)
